# baseline (speedup 1.0000x reference)
import functools

import jax
import jax.numpy as jnp
from jax import lax
from jax.experimental import pallas as pl
from jax.experimental.pallas import tpu as pltpu
from jax.experimental.pallas import tpu_sc as plsc

N = 10000
NP = 10240
E = 160000
G = 64
D_IN = 256
H = 512
OUT = 128

NTILES = 16
NCORES = 2
EB = 128
E_PAD = 163840
BATCHES_AGG = E_PAD // (NTILES * EB)
BATCHES_DEG = E_PAD // (NCORES * NTILES * EB)
ACC_ROWS = NP
STRIPE = NP // NTILES

_sc_mesh = plsc.VectorSubcoreMesh(core_axis_name="c", subcore_axis_name="s")


NBUF = 2
NG = 2
GI = BATCHES_AGG // NG


def _make_agg(tbl_planes, assigns):
  n_out = len(assigns)

  @functools.partial(
      pl.kernel,
      out_type=jax.ShapeDtypeStruct((n_out, NP, 128), jnp.float32),
      mesh=_sc_mesh,
      scratch_types=[
          pltpu.VMEM_SHARED((ACC_ROWS, 128), jnp.float32),
          pltpu.VMEM((GI, EB), jnp.int32),
          pltpu.VMEM((GI, EB), jnp.int32),
      ]
      + [pltpu.VMEM((EB, 128), jnp.float32)] * NBUF
      + [pltpu.SemaphoreType.DMA] * (2 * NBUF),
  )
  def agg(tbl_hbm, src_hbm, dst_hbm, out_hbm, acc, srcg, dstg, *bs):
    bufs = bs[:NBUF]
    gsems = bs[NBUF:2 * NBUF]
    ssems = bs[2 * NBUF:]
    core = lax.axis_index("c")
    sid = lax.axis_index("s")
    for acore, chunk, oplane in assigns:
      @pl.when(core == acore)
      def _():
        tbl = tbl_hbm.at[chunk]
        pltpu.sync_copy(tbl.at[pl.ds(sid * STRIPE, STRIPE)],
                        acc.at[pl.ds(sid * STRIPE, STRIPE)])
        plsc.subcore_barrier()

        for g in range(NG):
          pltpu.sync_copy(src_hbm.at[sid, pl.ds(g * GI, GI)], srcg)
          pltpu.sync_copy(dst_hbm.at[sid, pl.ds(g * GI, GI)], dstg)
          for b in range(NBUF):
            pltpu.async_copy(tbl.at[srcg.at[b]], bufs[b], gsems[b])

          @pl.loop(0, GI, step=NBUF)
          def _(j):
            for b in range(NBUF):
              jb = j + b
              pltpu.make_async_copy(tbl.at[srcg.at[jb]], bufs[b],
                                    gsems[b]).wait()
              pltpu.async_copy(bufs[b], acc.at[dstg.at[jb]], ssems[b],
                               add=True)
              jn = jb + NBUF

              @pl.when(jn < GI)
              def _():
                pltpu.make_async_copy(bufs[b], acc.at[dstg.at[jb]],
                                      ssems[b]).wait()
                pltpu.async_copy(tbl.at[srcg.at[jn]], bufs[b], gsems[b])

          for b in range(NBUF):
            pltpu.make_async_copy(bufs[b], acc.at[dstg.at[b]],
                                  ssems[b]).wait()

        plsc.subcore_barrier()
        pltpu.sync_copy(acc.at[pl.ds(sid * STRIPE, STRIPE)],
                        out_hbm.at[oplane, pl.ds(sid * STRIPE, STRIPE)])
        plsc.subcore_barrier()

  return agg


_agg2 = _make_agg(2, [(0, 0, 0), (1, 1, 1)])
_agg4a = _make_agg(4, [(0, 0, 0), (1, 2, 1)])
_agg4b = _make_agg(4, [(0, 1, 0), (1, 3, 1)])


@functools.partial(
    pl.kernel,
    out_type=jax.ShapeDtypeStruct((NCORES, NP, 128), jnp.float32),
    mesh=_sc_mesh,
    scratch_types=[
        pltpu.VMEM_SHARED((ACC_ROWS, 128), jnp.float32),
        pltpu.VMEM((BATCHES_DEG, EB), jnp.int32),
        pltpu.VMEM((EB, 128), jnp.float32),
    ] + [pltpu.SemaphoreType.DMA] * 4,
)
def _deg_sc(dst_hbm, ones_hbm, zeros_hbm, out_hbm, acc, dstb, onesb, *sems):
  core = lax.axis_index("c")
  sid = lax.axis_index("s")
  w = core * NTILES + sid
  pltpu.sync_copy(dst_hbm.at[w], dstb)
  pltpu.sync_copy(ones_hbm, onesb)
  pltpu.sync_copy(zeros_hbm.at[pl.ds(sid * STRIPE, STRIPE)],
                  acc.at[pl.ds(sid * STRIPE, STRIPE)])
  plsc.subcore_barrier()

  for b in range(4):
    pltpu.async_copy(onesb, acc.at[dstb.at[b]], sems[b], add=True)

  @pl.loop(0, BATCHES_DEG - 4, step=4)
  def _(j):
    for b in range(4):
      pltpu.make_async_copy(onesb, acc.at[dstb.at[j]], sems[b]).wait()
      pltpu.async_copy(onesb, acc.at[dstb.at[j + 4 + b]], sems[b], add=True)

  for b in range(4):
    pltpu.make_async_copy(onesb, acc.at[dstb.at[0]], sems[b]).wait()

  plsc.subcore_barrier()
  for c in range(NCORES):
    @pl.when(core == c)
    def _():
      pltpu.sync_copy(acc.at[pl.ds(sid * STRIPE, STRIPE)],
                      out_hbm.at[c, pl.ds(sid * STRIPE, STRIPE)])


NB = 1024
NSTEPS = NP // NB


def _ka_body(deg_ref, x_ref, dinv_ref, xs_ref):
  deg = deg_ref[0, :, 0] + deg_ref[1, :, 0] + 1.0
  dinv = lax.rsqrt(deg)[:, None]
  dinv_ref[...] = dinv
  xs = x_ref[...] * dinv
  for c in range(2):
    xs_ref[c] = xs[:, c * 128:(c + 1) * 128]


def _ka(degp, x):
  return pl.pallas_call(
      _ka_body,
      grid=(NSTEPS,),
      in_specs=[
          pl.BlockSpec((NCORES, NB, 128), lambda i: (0, i, 0)),
          pl.BlockSpec((NB, D_IN), lambda i: (i, 0)),
      ],
      out_specs=[
          pl.BlockSpec((NB, 1), lambda i: (i, 0)),
          pl.BlockSpec((2, NB, 128), lambda i: (0, i, 0)),
      ],
      out_shape=[
          jax.ShapeDtypeStruct((NP, 1), jnp.float32),
          jax.ShapeDtypeStruct((2, NP, 128), jnp.float32),
      ],
  )(degp, x)


def _k1_body(a_ref, dinv_ref, w1_ref, b1_ref, w2_ref, u_ref):
  dinv = dinv_ref[...]
  agg = jnp.concatenate([a_ref[0], a_ref[1]], axis=1)
  z = agg * dinv
  h = jnp.maximum(
      jnp.dot(z, w1_ref[...], preferred_element_type=jnp.float32) + b1_ref[...],
      0.0)
  u = jnp.dot(h, w2_ref[...], preferred_element_type=jnp.float32) * dinv
  for c in range(4):
    u_ref[c] = u[:, c * 128:(c + 1) * 128]


def _k1(aggx, dinv2, W1, b1, W2):
  return pl.pallas_call(
      _k1_body,
      grid=(NSTEPS,),
      in_specs=[
          pl.BlockSpec((2, NB, 128), lambda i: (0, i, 0)),
          pl.BlockSpec((NB, 1), lambda i: (i, 0)),
          pl.BlockSpec((D_IN, H), lambda i: (0, 0)),
          pl.BlockSpec((1, H), lambda i: (0, 0)),
          pl.BlockSpec((H, H), lambda i: (0, 0)),
      ],
      out_specs=pl.BlockSpec((4, NB, 128), lambda i: (0, i, 0)),
      out_shape=jax.ShapeDtypeStruct((4, NP, 128), jnp.float32),
  )(aggx, dinv2, W1, b1, W2)


def _k2a_body(a_ref, dinv_ref, b_ref, w_ref, p_ref):
  dinv = dinv_ref[...]
  agg = jnp.concatenate([a_ref[0], a_ref[1]], axis=1)
  h = jnp.maximum(agg * dinv + b_ref[...], 0.0)
  p_ref[...] = jnp.dot(h, w_ref[...], preferred_element_type=jnp.float32)


def _k2a(aggp, dinv2, bp, Wp):
  return pl.pallas_call(
      _k2a_body,
      grid=(NSTEPS,),
      in_specs=[
          pl.BlockSpec((2, NB, 128), lambda i: (0, i, 0)),
          pl.BlockSpec((NB, 1), lambda i: (i, 0)),
          pl.BlockSpec((1, 256), lambda i: (0, 0)),
          pl.BlockSpec((256, H), lambda i: (0, 0)),
      ],
      out_specs=pl.BlockSpec((NB, H), lambda i: (i, 0)),
      out_shape=jax.ShapeDtypeStruct((NP, H), jnp.float32),
  )(aggp, dinv2, bp, Wp)


def _k2b_body(a_ref, dinv_ref, b_ref, w_ref, p_ref, u_ref):
  dinv = dinv_ref[...]
  agg = jnp.concatenate([a_ref[0], a_ref[1]], axis=1)
  h = jnp.maximum(agg * dinv + b_ref[...], 0.0)
  u = (p_ref[...]
       + jnp.dot(h, w_ref[...], preferred_element_type=jnp.float32)) * dinv
  for c in range(4):
    u_ref[c] = u[:, c * 128:(c + 1) * 128]


def _k2b(aggp, dinv2, bp, Wp, part):
  return pl.pallas_call(
      _k2b_body,
      grid=(NSTEPS,),
      in_specs=[
          pl.BlockSpec((2, NB, 128), lambda i: (0, i, 0)),
          pl.BlockSpec((NB, 1), lambda i: (i, 0)),
          pl.BlockSpec((1, 256), lambda i: (0, 0)),
          pl.BlockSpec((256, H), lambda i: (0, 0)),
          pl.BlockSpec((NB, H), lambda i: (i, 0)),
      ],
      out_specs=pl.BlockSpec((4, NB, 128), lambda i: (0, i, 0)),
      out_shape=jax.ShapeDtypeStruct((4, NP, 128), jnp.float32),
  )(aggp, dinv2, bp, Wp, part)


def _k3a_body(a_ref, dinv_ref, b_ref, batch_ref, sums_ref):
  dinv = dinv_ref[...]
  agg = jnp.concatenate([a_ref[0], a_ref[1]], axis=1)
  h = jnp.maximum(agg * dinv + b_ref[...], 0.0)
  gid = lax.broadcasted_iota(jnp.int32, (G, NB), 0)
  sel = (gid == batch_ref[...].reshape(1, NB)).astype(jnp.float32)

  @pl.when(pl.program_id(0) == 0)
  def _():
    sums_ref[...] = jnp.zeros_like(sums_ref)

  sums_ref[...] += jnp.dot(sel, h, preferred_element_type=jnp.float32)


def _k3a(aggp, dinv2, bp, batchr):
  return pl.pallas_call(
      _k3a_body,
      grid=(NSTEPS,),
      in_specs=[
          pl.BlockSpec((2, NB, 128), lambda i: (0, i, 0)),
          pl.BlockSpec((NB, 1), lambda i: (i, 0)),
          pl.BlockSpec((1, 256), lambda i: (0, 0)),
          pl.BlockSpec((NB, 1), lambda i: (i, 0)),
      ],
      out_specs=pl.BlockSpec((G, 256), lambda i: (0, 0)),
      out_shape=jax.ShapeDtypeStruct((G, 256), jnp.float32),
  )(aggp, dinv2, bp, batchr)


def _k3b_body(a_ref, dinv_ref, b_ref, batch_ref, sums_ref, cnt_ref):
  dinv = dinv_ref[...]
  agg = jnp.concatenate([a_ref[0], a_ref[1]], axis=1)
  h = jnp.maximum(agg * dinv + b_ref[...], 0.0)
  gid = lax.broadcasted_iota(jnp.int32, (G, NB), 0)
  sel = (gid == batch_ref[...].reshape(1, NB)).astype(jnp.float32)

  @pl.when(pl.program_id(0) == 0)
  def _():
    sums_ref[...] = jnp.zeros_like(sums_ref)
    cnt_ref[...] = jnp.zeros_like(cnt_ref)

  sums_ref[...] += jnp.dot(sel, h, preferred_element_type=jnp.float32)
  cnt_ref[...] += jnp.sum(sel, axis=1, keepdims=True)


def _k3b(aggp, dinv2, bp, batchr):
  return pl.pallas_call(
      _k3b_body,
      grid=(NSTEPS,),
      in_specs=[
          pl.BlockSpec((2, NB, 128), lambda i: (0, i, 0)),
          pl.BlockSpec((NB, 1), lambda i: (i, 0)),
          pl.BlockSpec((1, 256), lambda i: (0, 0)),
          pl.BlockSpec((NB, 1), lambda i: (i, 0)),
      ],
      out_specs=[
          pl.BlockSpec((G, 256), lambda i: (0, 0)),
          pl.BlockSpec((G, 1), lambda i: (0, 0)),
      ],
      out_shape=[
          jax.ShapeDtypeStruct((G, 256), jnp.float32),
          jax.ShapeDtypeStruct((G, 1), jnp.float32),
      ],
  )(aggp, dinv2, bp, batchr)


def _k4_body(sa_ref, sb_ref, cnt_ref, wl1_ref, bl1_ref, wl2_ref, bl2_ref,
             wl3_ref, bl3_ref, out_ref):
  sums = jnp.concatenate([sa_ref[...], sb_ref[...]], axis=1)
  pooled = sums / jnp.maximum(cnt_ref[...], 1.0)
  z = jnp.maximum(
      jnp.dot(pooled, wl1_ref[...], preferred_element_type=jnp.float32)
      + bl1_ref[...], 0.0)
  z = jnp.maximum(
      jnp.dot(z, wl2_ref[...], preferred_element_type=jnp.float32)
      + bl2_ref[...], 0.0)
  out_ref[...] = (jnp.dot(z, wl3_ref[...], preferred_element_type=jnp.float32)
                  + bl3_ref[...])


def _k4(sums_a, sums_b, cnt, Wl1p, bl1, Wl2, bl2, Wl3, bl3):
  return pl.pallas_call(
      _k4_body,
      out_shape=jax.ShapeDtypeStruct((G, OUT), jnp.float32),
  )(sums_a, sums_b, cnt, Wl1p, bl1, Wl2, bl2, Wl3, bl3)


def kernel(x, edge_index, batch, W1, b1, W2, b2, W3, b3,
           Wl1, bl1, Wl2, bl2, Wl3, bl3):
  src = edge_index[0]
  dst = edge_index[1]
  pad = E_PAD - E
  srcp = jnp.concatenate([src, jnp.zeros((pad,), jnp.int32)])
  dstp = jnp.concatenate([dst, jnp.full((pad,), N, jnp.int32)])
  src3 = srcp.reshape(NTILES, BATCHES_AGG, EB)
  dst3 = dstp.reshape(NTILES, BATCHES_AGG, EB)
  dst3d = dstp.reshape(NCORES * NTILES, BATCHES_DEG, EB)
  ones16 = jnp.ones((EB, 128), jnp.float32)
  zeros16 = jnp.zeros((ACC_ROWS, 128), jnp.float32)
  xp = jnp.pad(x, ((0, NP - N), (0, 0)))
  batchp = jnp.concatenate([batch, jnp.full((NP - N,), G, jnp.int32)])

  b2a = jnp.concatenate([b2[0:128], b2[256:384]]).reshape(1, 256)
  b2b = jnp.concatenate([b2[128:256], b2[384:512]]).reshape(1, 256)
  W3a = jnp.concatenate([W3[0:128], W3[256:384]], axis=0)
  W3b = jnp.concatenate([W3[128:256], W3[384:512]], axis=0)
  b3a = jnp.concatenate([b3[0:128], b3[256:384]]).reshape(1, 256)
  b3b = jnp.concatenate([b3[128:256], b3[384:512]]).reshape(1, 256)
  Wl1p = jnp.concatenate(
      [Wl1[0:128], Wl1[256:384], Wl1[128:256], Wl1[384:512]], axis=0)
  batchr = batchp.reshape(NP, 1)

  degp = _deg_sc(dst3d, ones16, zeros16)
  dinv2, xs = _ka(degp, xp)
  aggx = _agg2(xs, src3, dst3)
  u2 = _k1(aggx, dinv2, W1, b1.reshape(1, H), W2)
  a2a = _agg4a(u2, src3, dst3)
  a2b = _agg4b(u2, src3, dst3)
  part = _k2a(a2a, dinv2, b2a, W3a)
  u3 = _k2b(a2b, dinv2, b2b, W3b, part)
  a3a = _agg4a(u3, src3, dst3)
  a3b = _agg4b(u3, src3, dst3)
  sums_a = _k3a(a3a, dinv2, b3a, batchr)
  sums_b, cnt = _k3b(a3b, dinv2, b3b, batchr)
  return _k4(sums_a, sums_b, cnt, Wl1p, bl1.reshape(1, H), Wl2,
             bl2.reshape(1, 256), Wl3, bl3.reshape(1, OUT))


# speedup vs baseline: 1.0247x; 1.0247x over previous
import functools

import jax
import jax.numpy as jnp
from jax import lax
from jax.experimental import pallas as pl
from jax.experimental.pallas import tpu as pltpu
from jax.experimental.pallas import tpu_sc as plsc

N = 10000
NP = 10240
E = 160000
G = 64
D_IN = 256
H = 512
OUT = 128

NTILES = 16
NCORES = 2
EB = 128
E_PAD = 163840
BATCHES_AGG = E_PAD // (NTILES * EB)
BATCHES_DEG = E_PAD // (NCORES * NTILES * EB)
ACC_ROWS = NP
STRIPE = NP // NTILES

_sc_mesh = plsc.VectorSubcoreMesh(core_axis_name="c", subcore_axis_name="s")


NBUF = 2
NG = 2
GI = BATCHES_AGG // NG


def _make_agg(C):
  cpc = C // NCORES

  @functools.partial(
      pl.kernel,
      out_type=jax.ShapeDtypeStruct((C, NP, 128), jnp.float32),
      mesh=_sc_mesh,
      scratch_types=[
          pltpu.VMEM_SHARED((ACC_ROWS, 128), jnp.float32),
          pltpu.VMEM((GI, EB), jnp.int32),
          pltpu.VMEM((GI, EB), jnp.int32),
      ]
      + [pltpu.VMEM((EB, 128), jnp.float32)] * NBUF
      + [pltpu.SemaphoreType.DMA] * (2 * NBUF),
  )
  def agg(tbl_hbm, src_hbm, dst_hbm, out_hbm, acc, srcg, dstg, *bs):
    bufs = bs[:NBUF]
    gsems = bs[NBUF:2 * NBUF]
    ssems = bs[2 * NBUF:]
    core = lax.axis_index("c")
    sid = lax.axis_index("s")
    for chunk in range(C):
      @pl.when(core == chunk // cpc)
      def _():
        tbl = tbl_hbm.at[chunk]
        pltpu.sync_copy(tbl.at[pl.ds(sid * STRIPE, STRIPE)],
                        acc.at[pl.ds(sid * STRIPE, STRIPE)])
        plsc.subcore_barrier()

        for g in range(NG):
          pltpu.sync_copy(src_hbm.at[sid, pl.ds(g * GI, GI)], srcg)
          pltpu.sync_copy(dst_hbm.at[sid, pl.ds(g * GI, GI)], dstg)
          for b in range(NBUF):
            pltpu.async_copy(tbl.at[srcg.at[b]], bufs[b], gsems[b])

          @pl.loop(0, GI, step=NBUF)
          def _(j):
            for b in range(NBUF):
              jb = j + b
              pltpu.make_async_copy(tbl.at[srcg.at[jb]], bufs[b],
                                    gsems[b]).wait()
              pltpu.async_copy(bufs[b], acc.at[dstg.at[jb]], ssems[b],
                               add=True)
              jn = jb + NBUF

              @pl.when(jn < GI)
              def _():
                pltpu.make_async_copy(bufs[b], acc.at[dstg.at[jb]],
                                      ssems[b]).wait()
                pltpu.async_copy(tbl.at[srcg.at[jn]], bufs[b], gsems[b])

          for b in range(NBUF):
            pltpu.make_async_copy(bufs[b], acc.at[dstg.at[b]],
                                  ssems[b]).wait()

        plsc.subcore_barrier()
        pltpu.sync_copy(acc.at[pl.ds(sid * STRIPE, STRIPE)],
                        out_hbm.at[chunk, pl.ds(sid * STRIPE, STRIPE)])
        plsc.subcore_barrier()

  return agg


_agg2 = _make_agg(2)
_agg4 = _make_agg(4)


@functools.partial(
    pl.kernel,
    out_type=jax.ShapeDtypeStruct((NCORES, NP, 128), jnp.float32),
    mesh=_sc_mesh,
    scratch_types=[
        pltpu.VMEM_SHARED((ACC_ROWS, 128), jnp.float32),
        pltpu.VMEM((BATCHES_DEG, EB), jnp.int32),
        pltpu.VMEM((EB, 128), jnp.float32),
    ] + [pltpu.SemaphoreType.DMA] * 4,
)
def _deg_sc(dst_hbm, ones_hbm, zeros_hbm, out_hbm, acc, dstb, onesb, *sems):
  core = lax.axis_index("c")
  sid = lax.axis_index("s")
  w = core * NTILES + sid
  pltpu.sync_copy(dst_hbm.at[w], dstb)
  pltpu.sync_copy(ones_hbm, onesb)
  pltpu.sync_copy(zeros_hbm.at[pl.ds(sid * STRIPE, STRIPE)],
                  acc.at[pl.ds(sid * STRIPE, STRIPE)])
  plsc.subcore_barrier()

  for b in range(4):
    pltpu.async_copy(onesb, acc.at[dstb.at[b]], sems[b], add=True)

  @pl.loop(0, BATCHES_DEG - 4, step=4)
  def _(j):
    for b in range(4):
      pltpu.make_async_copy(onesb, acc.at[dstb.at[j]], sems[b]).wait()
      pltpu.async_copy(onesb, acc.at[dstb.at[j + 4 + b]], sems[b], add=True)

  for b in range(4):
    pltpu.make_async_copy(onesb, acc.at[dstb.at[0]], sems[b]).wait()

  plsc.subcore_barrier()
  for c in range(NCORES):
    @pl.when(core == c)
    def _():
      pltpu.sync_copy(acc.at[pl.ds(sid * STRIPE, STRIPE)],
                      out_hbm.at[c, pl.ds(sid * STRIPE, STRIPE)])


NB = 1024
NSTEPS = NP // NB


def _ka_body(deg_ref, x_ref, dinv_ref, xs_ref):
  deg = deg_ref[0, :, 0] + deg_ref[1, :, 0] + 1.0
  dinv = lax.rsqrt(deg)[:, None]
  dinv_ref[...] = dinv
  xs = x_ref[...] * dinv
  for c in range(2):
    xs_ref[c] = xs[:, c * 128:(c + 1) * 128]


def _ka(degp, x):
  return pl.pallas_call(
      _ka_body,
      grid=(NSTEPS,),
      in_specs=[
          pl.BlockSpec((NCORES, NB, 128), lambda i: (0, i, 0)),
          pl.BlockSpec((NB, D_IN), lambda i: (i, 0)),
      ],
      out_specs=[
          pl.BlockSpec((NB, 1), lambda i: (i, 0)),
          pl.BlockSpec((2, NB, 128), lambda i: (0, i, 0)),
      ],
      out_shape=[
          jax.ShapeDtypeStruct((NP, 1), jnp.float32),
          jax.ShapeDtypeStruct((2, NP, 128), jnp.float32),
      ],
  )(degp, x)


def _k1_body(a_ref, dinv_ref, w1_ref, b1_ref, w2_ref, u_ref):
  dinv = dinv_ref[...]
  agg = jnp.concatenate([a_ref[0], a_ref[1]], axis=1)
  z = agg * dinv
  h = jnp.maximum(
      jnp.dot(z, w1_ref[...], preferred_element_type=jnp.float32) + b1_ref[...],
      0.0)
  u = jnp.dot(h, w2_ref[...], preferred_element_type=jnp.float32) * dinv
  for c in range(4):
    u_ref[c] = u[:, c * 128:(c + 1) * 128]


def _k1(aggx, dinv2, W1, b1, W2):
  return pl.pallas_call(
      _k1_body,
      grid=(NSTEPS,),
      in_specs=[
          pl.BlockSpec((2, NB, 128), lambda i: (0, i, 0)),
          pl.BlockSpec((NB, 1), lambda i: (i, 0)),
          pl.BlockSpec((D_IN, H), lambda i: (0, 0)),
          pl.BlockSpec((1, H), lambda i: (0, 0)),
          pl.BlockSpec((H, H), lambda i: (0, 0)),
      ],
      out_specs=pl.BlockSpec((4, NB, 128), lambda i: (0, i, 0)),
      out_shape=jax.ShapeDtypeStruct((4, NP, 128), jnp.float32),
  )(aggx, dinv2, W1, b1, W2)


def _k2_body(a_ref, dinv_ref, b_ref, w_ref, u_ref):
  dinv = dinv_ref[...]
  agg = jnp.concatenate([a_ref[0], a_ref[1], a_ref[2], a_ref[3]], axis=1)
  h = jnp.maximum(agg * dinv + b_ref[...], 0.0)
  u = jnp.dot(h, w_ref[...], preferred_element_type=jnp.float32) * dinv
  for c in range(4):
    u_ref[c] = u[:, c * 128:(c + 1) * 128]


def _k2(agg, dinv2, b, W):
  return pl.pallas_call(
      _k2_body,
      grid=(NSTEPS,),
      in_specs=[
          pl.BlockSpec((4, NB, 128), lambda i: (0, i, 0)),
          pl.BlockSpec((NB, 1), lambda i: (i, 0)),
          pl.BlockSpec((1, H), lambda i: (0, 0)),
          pl.BlockSpec((H, H), lambda i: (0, 0)),
      ],
      out_specs=pl.BlockSpec((4, NB, 128), lambda i: (0, i, 0)),
      out_shape=jax.ShapeDtypeStruct((4, NP, 128), jnp.float32),
  )(agg, dinv2, b, W)


def _k3_body(a_ref, dinv_ref, b_ref, batch_ref, sums_ref, cnt_ref):
  dinv = dinv_ref[...]
  agg = jnp.concatenate([a_ref[0], a_ref[1], a_ref[2], a_ref[3]], axis=1)
  h = jnp.maximum(agg * dinv + b_ref[...], 0.0)
  gid = lax.broadcasted_iota(jnp.int32, (G, NB), 0)
  sel = (gid == batch_ref[...].reshape(1, NB)).astype(jnp.float32)

  @pl.when(pl.program_id(0) == 0)
  def _():
    sums_ref[...] = jnp.zeros_like(sums_ref)
    cnt_ref[...] = jnp.zeros_like(cnt_ref)

  sums_ref[...] += jnp.dot(sel, h, preferred_element_type=jnp.float32)
  cnt_ref[...] += jnp.sum(sel, axis=1, keepdims=True)


def _k3(agg, dinv2, b3, batchr):
  return pl.pallas_call(
      _k3_body,
      grid=(NSTEPS,),
      in_specs=[
          pl.BlockSpec((4, NB, 128), lambda i: (0, i, 0)),
          pl.BlockSpec((NB, 1), lambda i: (i, 0)),
          pl.BlockSpec((1, H), lambda i: (0, 0)),
          pl.BlockSpec((NB, 1), lambda i: (i, 0)),
      ],
      out_specs=[
          pl.BlockSpec((G, H), lambda i: (0, 0)),
          pl.BlockSpec((G, 1), lambda i: (0, 0)),
      ],
      out_shape=[
          jax.ShapeDtypeStruct((G, H), jnp.float32),
          jax.ShapeDtypeStruct((G, 1), jnp.float32),
      ],
  )(agg, dinv2, b3, batchr)


def _k4_body(sums_ref, cnt_ref, wl1_ref, bl1_ref, wl2_ref, bl2_ref,
             wl3_ref, bl3_ref, out_ref):
  pooled = sums_ref[...] / jnp.maximum(cnt_ref[...], 1.0)
  z = jnp.maximum(
      jnp.dot(pooled, wl1_ref[...], preferred_element_type=jnp.float32)
      + bl1_ref[...], 0.0)
  z = jnp.maximum(
      jnp.dot(z, wl2_ref[...], preferred_element_type=jnp.float32)
      + bl2_ref[...], 0.0)
  out_ref[...] = (jnp.dot(z, wl3_ref[...], preferred_element_type=jnp.float32)
                  + bl3_ref[...])


def _k4(sums, cnt, Wl1, bl1, Wl2, bl2, Wl3, bl3):
  return pl.pallas_call(
      _k4_body,
      out_shape=jax.ShapeDtypeStruct((G, OUT), jnp.float32),
  )(sums, cnt, Wl1, bl1, Wl2, bl2, Wl3, bl3)


def kernel(x, edge_index, batch, W1, b1, W2, b2, W3, b3,
           Wl1, bl1, Wl2, bl2, Wl3, bl3):
  src = edge_index[0]
  dst = edge_index[1]
  pad = E_PAD - E
  srcp = jnp.concatenate([src, jnp.zeros((pad,), jnp.int32)])
  dstp = jnp.concatenate([dst, jnp.full((pad,), N, jnp.int32)])
  src3 = srcp.reshape(NTILES, BATCHES_AGG, EB)
  dst3 = dstp.reshape(NTILES, BATCHES_AGG, EB)
  dst3d = dstp.reshape(NCORES * NTILES, BATCHES_DEG, EB)
  ones16 = jnp.ones((EB, 128), jnp.float32)
  zeros16 = jnp.zeros((ACC_ROWS, 128), jnp.float32)
  xp = jnp.pad(x, ((0, NP - N), (0, 0)))
  batchp = jnp.concatenate([batch, jnp.full((NP - N,), G, jnp.int32)])

  degp = _deg_sc(dst3d, ones16, zeros16)
  dinv2, xs = _ka(degp, xp)
  aggx = _agg2(xs, src3, dst3)
  u2 = _k1(aggx, dinv2, W1, b1.reshape(1, H), W2)
  agg2 = _agg4(u2, src3, dst3)
  u3 = _k2(agg2, dinv2, b2.reshape(1, H), W3)
  agg3 = _agg4(u3, src3, dst3)
  sums, cnt = _k3(agg3, dinv2, b3.reshape(1, H), batchp.reshape(NP, 1))
  return _k4(sums, cnt, Wl1, bl1.reshape(1, H), Wl2, bl2.reshape(1, 256),
             Wl3, bl3.reshape(1, OUT))
